# baseline (speedup 1.0000x reference)
import jax
import jax.numpy as jnp
from jax.experimental import pallas as pl
from jax.experimental.pallas import tpu as pltpu

_B, _N, _M, _C = 16, 20000, 100, 80
_MP = 128
_L_COORD, _L_OBJ, _L_NOOBJ, _L_CLS = 5.0, 1.0, 0.5, 1.0


def _loss_kernel(bbox_ref, obj_ref, cls_ref, gtb_ref, lbl_ref,
                 tot_ref, bb_ref, ob_ref, cl_ref):
    px, py = bbox_ref[0][:, : _M], bbox_ref[1][:, : _M]
    pw, ph = bbox_ref[2][:, : _M], bbox_ref[3][:, : _M]
    gx, gy, gw, gh = gtb_ref[0], gtb_ref[1], gtb_ref[2], gtb_ref[3]
    px1, px2 = px - pw * 0.5, px + pw * 0.5
    py1, py2 = py - ph * 0.5, py + ph * 0.5
    gx1, gx2 = gx - gw * 0.5, gx + gw * 0.5
    gy1, gy2 = gy - gh * 0.5, gy + gh * 0.5
    iw = jnp.maximum(jnp.minimum(px2, gx2) - jnp.maximum(px1, gx1), 0.0)
    ih = jnp.maximum(jnp.minimum(py2, gy2) - jnp.maximum(py1, gy1), 0.0)
    inter = iw * ih
    union = (px2 - px1) * (py2 - py1) + (gx2 - gx1) * (gy2 - gy1) - inter
    iou = inter / (union + 1e-07)
    ew = jnp.maximum(px2, gx2) - jnp.minimum(px1, gx1)
    eh = jnp.maximum(py2, gy2) - jnp.minimum(py1, gy1)
    enclose = ew * eh
    giou = 1.0 - (iou - (enclose - union) / (enclose + 1e-07))
    loss_bbox = jnp.sum(giou) * (_L_COORD / (_B * _M))

    x = obj_ref[...]
    t = jnp.log1p(jnp.exp(-jnp.abs(x)))
    all_sum = jnp.sum(t + jnp.maximum(x, 0.0))
    xs = x[:, : _M]
    ts = t[:, : _M]
    pos_sum = jnp.sum(ts + jnp.maximum(-xs, 0.0))
    over_sum = jnp.sum(ts + jnp.maximum(xs, 0.0))
    loss_obj = pos_sum * (_L_OBJ / (_B * _M)) + (all_sum - over_sum) * (
        _L_NOOBJ / (_B * (_N - _M))
    )

    z = cls_ref[:, : _M, :]
    m = jnp.max(z, axis=-1)
    lse = m + jnp.log(jnp.sum(jnp.exp(z - m[:, :, None]), axis=-1))
    lab = lbl_ref[...]
    cls_iota = jax.lax.broadcasted_iota(jnp.int32, (_B, _M, _C), 2)
    z_lab = jnp.sum(jnp.where(cls_iota == lab[:, :, None], z, 0.0), axis=-1)
    nll = lse - z_lab
    loss_cls = jnp.sum(nll) * (_L_CLS / (_B * _M))

    tot_ref[0] = loss_bbox + loss_obj + loss_cls
    bb_ref[0] = loss_bbox
    ob_ref[0] = loss_obj
    cl_ref[0] = loss_cls


def kernel(pred_bbox, pred_obj, pred_cls, gt_boxes, gt_labels):
    bbox_t = jnp.transpose(pred_bbox[:, :_MP, :], (2, 0, 1))
    gt_t = jnp.transpose(gt_boxes, (2, 0, 1))
    lbl = gt_labels.astype(jnp.int32)
    scalar = jax.ShapeDtypeStruct((1,), jnp.float32)
    smem = pl.BlockSpec(memory_space=pltpu.SMEM)
    tot, bb, ob, cl = pl.pallas_call(
        _loss_kernel,
        out_shape=(scalar, scalar, scalar, scalar),
        grid=(1,),
        in_specs=[
            pl.BlockSpec((4, _B, _MP), lambda i: (0, 0, 0)),
            pl.BlockSpec((_B, _N), lambda i: (0, 0)),
            pl.BlockSpec((_B, _MP, _C), lambda i: (0, 0, 0)),
            pl.BlockSpec((4, _B, _M), lambda i: (0, 0, 0)),
            pl.BlockSpec((_B, _M), lambda i: (0, 0)),
        ],
        out_specs=(smem, smem, smem, smem),
    )(bbox_t, pred_obj, pred_cls, gt_t, lbl)
    return (tot[0], bb[0], ob[0], cl[0])


# speedup vs baseline: 11.7615x; 11.7615x over previous
import jax
import jax.numpy as jnp
from jax.experimental import pallas as pl
from jax.experimental.pallas import tpu as pltpu

_B, _N, _M, _C = 16, 20000, 100, 80
_MP = 128
_L_COORD, _L_OBJ, _L_NOOBJ, _L_CLS = 5.0, 1.0, 0.5, 1.0


def _loss_kernel(bbox_ref, obj_hbm, cls_hbm, gtb_ref, lbl_ref,
                 tot_ref, bb_ref, ob_ref, cl_ref,
                 obj_vmem, cls_vmem, obj_sem, cls_sem):
    obj_cp = pltpu.make_async_copy(obj_hbm, obj_vmem, obj_sem)
    cls_cp = pltpu.make_async_copy(cls_hbm, cls_vmem, cls_sem)
    obj_cp.start()
    cls_cp.start()

    px, py = bbox_ref[0][:, : _M], bbox_ref[1][:, : _M]
    pw, ph = bbox_ref[2][:, : _M], bbox_ref[3][:, : _M]
    gx, gy, gw, gh = gtb_ref[0], gtb_ref[1], gtb_ref[2], gtb_ref[3]
    px1, px2 = px - pw * 0.5, px + pw * 0.5
    py1, py2 = py - ph * 0.5, py + ph * 0.5
    gx1, gx2 = gx - gw * 0.5, gx + gw * 0.5
    gy1, gy2 = gy - gh * 0.5, gy + gh * 0.5
    iw = jnp.maximum(jnp.minimum(px2, gx2) - jnp.maximum(px1, gx1), 0.0)
    ih = jnp.maximum(jnp.minimum(py2, gy2) - jnp.maximum(py1, gy1), 0.0)
    inter = iw * ih
    union = (px2 - px1) * (py2 - py1) + (gx2 - gx1) * (gy2 - gy1) - inter
    iou = inter / (union + 1e-07)
    ew = jnp.maximum(px2, gx2) - jnp.minimum(px1, gx1)
    eh = jnp.maximum(py2, gy2) - jnp.minimum(py1, gy1)
    enclose = ew * eh
    giou = 1.0 - (iou - (enclose - union) / (enclose + 1e-07))
    loss_bbox = jnp.sum(giou) * (_L_COORD / (_B * _M))

    cls_cp.wait()
    z = cls_vmem[:, : _M, :]
    m = jnp.max(z, axis=-1)
    lse = m + jnp.log(jnp.sum(jnp.exp(z - m[:, :, None]), axis=-1))
    lab = lbl_ref[...]
    cls_iota = jax.lax.broadcasted_iota(jnp.int32, (_B, _M, _C), 2)
    z_lab = jnp.sum(jnp.where(cls_iota == lab[:, :, None], z, 0.0), axis=-1)
    nll = lse - z_lab
    loss_cls = jnp.sum(nll) * (_L_CLS / (_B * _M))

    obj_cp.wait()
    x = obj_vmem[...]
    t = jnp.log1p(jnp.exp(-jnp.abs(x)))
    all_sum = jnp.sum(t + jnp.maximum(x, 0.0))
    xs = x[:, : _M]
    ts = t[:, : _M]
    pos_sum = jnp.sum(ts + jnp.maximum(-xs, 0.0))
    over_sum = jnp.sum(ts + jnp.maximum(xs, 0.0))
    loss_obj = pos_sum * (_L_OBJ / (_B * _M)) + (all_sum - over_sum) * (
        _L_NOOBJ / (_B * (_N - _M))
    )

    tot_ref[0] = loss_bbox + loss_obj + loss_cls
    bb_ref[0] = loss_bbox
    ob_ref[0] = loss_obj
    cl_ref[0] = loss_cls


def kernel(pred_bbox, pred_obj, pred_cls, gt_boxes, gt_labels):
    bbox_t = jnp.transpose(pred_bbox[:, :_MP, :], (2, 0, 1))
    gt_t = jnp.transpose(gt_boxes, (2, 0, 1))
    cls_s = pred_cls[:, :_MP, :]
    lbl = gt_labels.astype(jnp.int32)
    scalar = jax.ShapeDtypeStruct((1,), jnp.float32)
    smem = pl.BlockSpec(memory_space=pltpu.SMEM)
    tot, bb, ob, cl = pl.pallas_call(
        _loss_kernel,
        out_shape=(scalar, scalar, scalar, scalar),
        grid=(1,),
        in_specs=[
            pl.BlockSpec((4, _B, _MP), lambda i: (0, 0, 0)),
            pl.BlockSpec(memory_space=pl.ANY),
            pl.BlockSpec(memory_space=pl.ANY),
            pl.BlockSpec((4, _B, _M), lambda i: (0, 0, 0)),
            pl.BlockSpec((_B, _M), lambda i: (0, 0)),
        ],
        out_specs=(smem, smem, smem, smem),
        scratch_shapes=[
            pltpu.VMEM((_B, _N), jnp.float32),
            pltpu.VMEM((_B, _MP, _C), jnp.float32),
            pltpu.SemaphoreType.DMA,
            pltpu.SemaphoreType.DMA,
        ],
    )(bbox_t, pred_obj, cls_s, gt_t, lbl)
    return (tot[0], bb[0], ob[0], cl[0])


# speedup vs baseline: 13.3286x; 1.1332x over previous
import jax
import jax.numpy as jnp
from jax.experimental import pallas as pl
from jax.experimental.pallas import tpu as pltpu

_B, _N, _M, _C = 16, 20000, 100, 80
_MP = 104
_L_COORD, _L_OBJ, _L_NOOBJ, _L_CLS = 5.0, 1.0, 0.5, 1.0


def _loss_kernel(bbox_ref, obj_ref, cls_ref, gtb_ref, lbl_ref,
                 tot_ref, bb_ref, ob_ref, cl_ref):
    px, py = bbox_ref[0][:, : _M], bbox_ref[1][:, : _M]
    pw, ph = bbox_ref[2][:, : _M], bbox_ref[3][:, : _M]
    gx, gy, gw, gh = gtb_ref[0], gtb_ref[1], gtb_ref[2], gtb_ref[3]
    px1, px2 = px - pw * 0.5, px + pw * 0.5
    py1, py2 = py - ph * 0.5, py + ph * 0.5
    gx1, gx2 = gx - gw * 0.5, gx + gw * 0.5
    gy1, gy2 = gy - gh * 0.5, gy + gh * 0.5
    iw = jnp.maximum(jnp.minimum(px2, gx2) - jnp.maximum(px1, gx1), 0.0)
    ih = jnp.maximum(jnp.minimum(py2, gy2) - jnp.maximum(py1, gy1), 0.0)
    inter = iw * ih
    union = (px2 - px1) * (py2 - py1) + (gx2 - gx1) * (gy2 - gy1) - inter
    iou = inter / (union + 1e-07)
    ew = jnp.maximum(px2, gx2) - jnp.minimum(px1, gx1)
    eh = jnp.maximum(py2, gy2) - jnp.minimum(py1, gy1)
    enclose = ew * eh
    giou = 1.0 - (iou - (enclose - union) / (enclose + 1e-07))
    loss_bbox = jnp.sum(giou) * (_L_COORD / (_B * _M))

    x = obj_ref[...]
    t = jnp.log1p(jnp.exp(-jnp.abs(x)))
    all_sum = jnp.sum(t + jnp.maximum(x, 0.0))
    xs = x[:, : _M]
    ts = t[:, : _M]
    pos_sum = jnp.sum(ts + jnp.maximum(-xs, 0.0))
    over_sum = jnp.sum(ts + jnp.maximum(xs, 0.0))
    loss_obj = pos_sum * (_L_OBJ / (_B * _M)) + (all_sum - over_sum) * (
        _L_NOOBJ / (_B * (_N - _M))
    )

    z = cls_ref[:, : _M, :]
    m = jnp.max(z, axis=-1)
    lse = m + jnp.log(jnp.sum(jnp.exp(z - m[:, :, None]), axis=-1))
    lab = lbl_ref[...]
    cls_iota = jax.lax.broadcasted_iota(jnp.int32, (_B, _M, _C), 2)
    z_lab = jnp.sum(jnp.where(cls_iota == lab[:, :, None], z, 0.0), axis=-1)
    nll = lse - z_lab
    loss_cls = jnp.sum(nll) * (_L_CLS / (_B * _M))

    tot_ref[0] = loss_bbox + loss_obj + loss_cls
    bb_ref[0] = loss_bbox
    ob_ref[0] = loss_obj
    cl_ref[0] = loss_cls


def kernel(pred_bbox, pred_obj, pred_cls, gt_boxes, gt_labels):
    bbox_t = jnp.transpose(pred_bbox[:, :_MP, :], (2, 0, 1))
    gt_t = jnp.transpose(gt_boxes, (2, 0, 1))
    cls_s = pred_cls[:, :_MP, :]
    lbl = gt_labels.astype(jnp.int32)
    scalar = jax.ShapeDtypeStruct((1,), jnp.float32)
    smem = pl.BlockSpec(memory_space=pltpu.SMEM)
    tot, bb, ob, cl = pl.pallas_call(
        _loss_kernel,
        out_shape=(scalar, scalar, scalar, scalar),
        grid=(1,),
        in_specs=[
            pl.BlockSpec((4, _B, _MP), lambda i: (0, 0, 0)),
            pl.BlockSpec((_B, _N), lambda i: (0, 0)),
            pl.BlockSpec((_B, _MP, _C), lambda i: (0, 0, 0)),
            pl.BlockSpec((4, _B, _M), lambda i: (0, 0, 0)),
            pl.BlockSpec((_B, _M), lambda i: (0, 0)),
        ],
        out_specs=(smem, smem, smem, smem),
    )(bbox_t, pred_obj, cls_s, gt_t, lbl)
    return (tot[0], bb[0], ob[0], cl[0])
